# Initial kernel scaffold; baseline (speedup 1.0000x reference)
import jax
import jax.numpy as jnp
from jax.experimental import pallas as pl


def kernel(h, ei, ea, W, b, gamma, beta):
    raise NotImplementedError("write your pallas kernel here")


# speedup vs baseline: 4.3546x; 4.3546x over previous
import functools

import jax
import jax.numpy as jnp
from jax import lax
from jax.experimental import pallas as pl
from jax.experimental.pallas import tpu as pltpu
from jax.experimental.pallas import tpu_sc as plsc

E = 16
E2 = E * E
NC = 2
NS = 16
NW = NC * NS
G = 128
SG = 40
SGS = 8


def _sc_gather(h, idx2):
    n_nodes = h.shape[0]
    mp = idx2.shape[0] * G
    per_w = mp // NW
    n_groups = per_w // G
    n_steps = n_groups // SG
    chunk = SG * G
    mesh = plsc.VectorSubcoreMesh(core_axis_name="c", subcore_axis_name="s")

    @functools.partial(
        pl.kernel,
        mesh=mesh,
        out_type=jax.ShapeDtypeStruct((mp, E), jnp.float32),
        scratch_types=[
            pltpu.VMEM((SG, G), jnp.int32),
            pltpu.VMEM((chunk, E), jnp.float32),
            pltpu.SemaphoreType.DMA,
        ],
        compiler_params=pltpu.CompilerParams(use_tc_tiling_on_sc=False),
    )
    def k(h_hbm, idx_hbm, out_hbm, idx_v, rows_v, sem):
        wid = lax.axis_index("s") * NC + lax.axis_index("c")
        base = wid * per_w
        gbase = wid * n_groups
        for t in range(n_steps):
            off = base + t * chunk
            pltpu.sync_copy(
                idx_hbm.at[pl.ds(gbase + t * SG, SG)],
                idx_v.at[...],
            )
            copies = [
                pltpu.async_copy(
                    h_hbm.at[idx_v.at[j]],
                    rows_v.at[pl.ds(j * G, G)],
                    sem,
                )
                for j in range(SG)
            ]
            for cp in copies:
                cp.wait()
            pltpu.sync_copy(rows_v, out_hbm.at[pl.ds(off, chunk)])

    del n_nodes
    return k(h, idx2)


def _sc_scatter(msg, idx2, zeros_hbm):
    n_nodes = zeros_hbm.shape[0]
    mp = idx2.shape[0] * G
    per_w = mp // NW
    n_groups = per_w // G
    n_steps = n_groups // SGS
    chunk = SGS * G
    rows_per_s = n_nodes // NS
    mesh = plsc.VectorSubcoreMesh(core_axis_name="c", subcore_axis_name="s")

    @functools.partial(
        pl.kernel,
        mesh=mesh,
        out_type=jax.ShapeDtypeStruct((NC, n_nodes, E), jnp.float32),
        scratch_types=[
            pltpu.VMEM_SHARED((n_nodes, E), jnp.float32),
            pltpu.VMEM((SGS, G), jnp.int32),
            pltpu.VMEM((chunk, E), jnp.float32),
        ],
        compiler_params=pltpu.CompilerParams(use_tc_tiling_on_sc=False),
    )
    def k(msg_hbm, idx_hbm, z_hbm, out_hbm, agg_sh, idx_v, upd_v):
        cid = lax.axis_index("c")
        sid = lax.axis_index("s")
        wid = sid * NC + cid
        r0 = sid * rows_per_s
        pltpu.sync_copy(
            z_hbm.at[pl.ds(r0, rows_per_s)],
            agg_sh.at[pl.ds(r0, rows_per_s)],
        )
        plsc.subcore_barrier()
        base = wid * per_w
        gbase = wid * n_groups
        for t in range(n_steps):
            off = base + t * chunk
            pltpu.sync_copy(idx_hbm.at[pl.ds(gbase + t * SGS, SGS)], idx_v.at[...])
            pltpu.sync_copy(msg_hbm.at[pl.ds(off, chunk)], upd_v)
            for j in range(SGS):
                pltpu.sync_copy(
                    upd_v.at[pl.ds(j * G, G)],
                    agg_sh.at[idx_v.at[j]],
                    add=True,
                )
        plsc.subcore_barrier()
        pltpu.sync_copy(
            agg_sh.at[pl.ds(r0, rows_per_s)],
            out_hbm.at[cid, pl.ds(r0, rows_per_s)],
        )

    return k(msg, idx2, zeros_hbm)


def _tc_msg(ea, hj, wt, b2, r_mat, s_mat, m_real, blk):
    mp = hj.shape[0]
    grid = mp // blk
    last_real = m_real // blk - 1

    def body(ea_ref, hj_ref, wt_ref, b_ref, r_ref, s_ref, out_ref):
        i = pl.program_id(0)
        t = jnp.dot(ea_ref[...], wt_ref[...],
                    preferred_element_type=jnp.float32) + b_ref[...]
        hjt = jnp.dot(hj_ref[...], r_ref[...],
                      preferred_element_type=jnp.float32)
        msg = jnp.dot(t * hjt, s_ref[...],
                      preferred_element_type=jnp.float32)
        row = i * blk + lax.broadcasted_iota(jnp.int32, (blk, E), 0)
        out_ref[...] = jnp.where(row < m_real, msg, 0.0)

    return pl.pallas_call(
        body,
        grid=(grid,),
        in_specs=[
            pl.BlockSpec((blk, E), lambda i: (jnp.minimum(i, last_real), 0)),
            pl.BlockSpec((blk, E), lambda i: (i, 0)),
            pl.BlockSpec((E, E2), lambda i: (0, 0)),
            pl.BlockSpec((1, E2), lambda i: (0, 0)),
            pl.BlockSpec((E, E2), lambda i: (0, 0)),
            pl.BlockSpec((E2, E), lambda i: (0, 0)),
        ],
        out_specs=pl.BlockSpec((blk, E), lambda i: (i, 0)),
        out_shape=jax.ShapeDtypeStruct((mp, E), jnp.float32),
    )(ea, hj, wt, b2, r_mat, s_mat)


def _tc_layernorm(h, p0, p1, gamma2, beta2, blk):
    n = h.shape[0]
    grid = n // blk

    def body(h_ref, a_ref, b_ref, g_ref, bt_ref, out_ref):
        y = h_ref[...] + a_ref[...] + b_ref[...]
        mu = jnp.mean(y, axis=1, keepdims=True)
        yc = y - mu
        var = jnp.mean(yc * yc, axis=1, keepdims=True)
        out_ref[...] = yc * lax.rsqrt(var + 1e-5) * g_ref[...] + bt_ref[...]

    return pl.pallas_call(
        body,
        grid=(grid,),
        in_specs=[
            pl.BlockSpec((blk, E), lambda i: (i, 0)),
            pl.BlockSpec((blk, E), lambda i: (i, 0)),
            pl.BlockSpec((blk, E), lambda i: (i, 0)),
            pl.BlockSpec((1, E), lambda i: (0, 0)),
            pl.BlockSpec((1, E), lambda i: (0, 0)),
        ],
        out_specs=pl.BlockSpec((blk, E), lambda i: (i, 0)),
        out_shape=jax.ShapeDtypeStruct((n, E), jnp.float32),
    )(h, p0, p1, gamma2, beta2)


def kernel(h, ei, ea, W, b, gamma, beta):
    n_nodes = h.shape[0]
    m = ei.shape[1]
    blk = 6400
    mp = NW * G * SG * ((m + NW * G * SG - 1) // (NW * G * SG))

    ei32 = ei.astype(jnp.int32)
    pad = mp - m
    pad_idx = (jnp.arange(pad, dtype=jnp.int32) * 97) % n_nodes
    ei0 = jnp.concatenate([ei32[0], pad_idx]).reshape(mp // G, G)
    ei1 = jnp.concatenate([ei32[1], pad_idx]).reshape(mp // G, G)

    wt = W.T
    b2 = b.reshape(1, E2)
    eye = jnp.eye(E, dtype=jnp.float32)
    r_mat = jnp.tile(eye, (1, E))
    s_mat = jnp.repeat(eye, E, axis=0)
    zeros = jnp.zeros((n_nodes, E), jnp.float32)

    hj = _sc_gather(h, ei1)
    msg = _tc_msg(ea, hj, wt, b2, r_mat, s_mat, m, blk)
    part = _sc_scatter(msg, ei0, zeros)
    return _tc_layernorm(h, part[0], part[1],
                         gamma.reshape(1, E), beta.reshape(1, E), blk=5000)
